# baseline (speedup 1.0000x reference)
import functools

import jax
import jax.numpy as jnp
from jax import lax
from jax.experimental import pallas as pl
from jax.experimental.pallas import tpu as pltpu
from jax.experimental.pallas import tpu_sc as plsc

NUM_EMB = 1000000
D = 64
BATCH = 16384
HIST = 20

NC, NS = 2, 16
NW = NC * NS
ROWS_W = BATCH // NW
RC = 32
CH = ROWS_W // RC
NB = 2


def _make_gather():
  mesh = plsc.VectorSubcoreMesh(core_axis_name="c", subcore_axis_name="s")

  @functools.partial(
      pl.kernel,
      out_type=jax.ShapeDtypeStruct((BATCH, HIST, D), jnp.float32),
      mesh=mesh,
      scratch_types=[
          pltpu.VMEM((ROWS_W, HIST), jnp.int32),
          pltpu.VMEM((NB, RC, HIST, D), jnp.float32),
          pltpu.SemaphoreType.DMA((NB,)),
          pltpu.SemaphoreType.DMA((NB,)),
      ],
      compiler_params=pltpu.CompilerParams(use_tc_tiling_on_sc=False),
  )
  def gather_kernel(idx_hbm, table_hbm, out_hbm, idx_v, bufs, gsem, osem):
    wid = lax.axis_index("s") * NC + lax.axis_index("c")
    base_r = wid * ROWS_W

    pltpu.sync_copy(idx_hbm.at[pl.ds(base_r, ROWS_W)], idx_v)

    def gathers(j, b):
      return [
          pltpu.make_async_copy(
              table_hbm.at[idx_v.at[j * RC + r]],
              bufs.at[b, r],
              gsem.at[b],
          )
          for r in range(RC)
      ]

    def out_copy(j, b):
      return pltpu.make_async_copy(
          bufs.at[b],
          out_hbm.at[pl.ds(base_r + j * RC, RC)],
          osem.at[b],
      )

    for b in range(NB):
      for g in gathers(b, b):
        g.start()

    @pl.loop(0, CH - NB, step=NB)
    def _main(j0):
      for b in range(NB):
        j = j0 + b
        for g in gathers(j, b):
          g.wait()
        out_copy(j, b).start()
        out_copy(j, b).wait()
        for g in gathers(j + NB, b):
          g.start()

    for b in range(NB):
      j = CH - NB + b
      for g in gathers(j, b):
        g.wait()
      out_copy(j, b).start()
      out_copy(j, b).wait()

  return gather_kernel


_gather = _make_gather()


@jax.jit
def kernel(input, table):
  return _gather(input, table)


# speedup vs baseline: 1.0373x; 1.0373x over previous
import functools

import jax
import jax.numpy as jnp
from jax import lax
from jax.experimental import pallas as pl
from jax.experimental.pallas import tpu as pltpu
from jax.experimental.pallas import tpu_sc as plsc

BATCH, HIST, D = 16384, 20, 64
NC, NS = 2, 16
NW = NC * NS
ROWS_W = BATCH // NW
RC = 16
CH = ROWS_W // RC
NB = 2


def _make():
  mesh = plsc.VectorSubcoreMesh(core_axis_name="c", subcore_axis_name="s")

  @functools.partial(
      pl.kernel,
      out_type=jax.ShapeDtypeStruct((BATCH, HIST, D), jnp.float32),
      mesh=mesh,
      scratch_types=[
          pltpu.VMEM((ROWS_W, HIST), jnp.int32),
          pltpu.VMEM((NB, RC, HIST, 2 * D), jnp.float32),
          pltpu.SemaphoreType.DMA((NB,)),
          pltpu.SemaphoreType.DMA((NB,)),
      ],
      compiler_params=pltpu.CompilerParams(use_tc_tiling_on_sc=False),
  )
  def k(idx_hbm, table_hbm, out_hbm, idx_v, bufs, gsem, osem):
    wid = lax.axis_index("s") * NC + lax.axis_index("c")
    base_r = wid * ROWS_W
    pltpu.sync_copy(idx_hbm.at[pl.ds(base_r, ROWS_W)], idx_v)
    def gathers(j, b):
      return [
          pltpu.make_async_copy(
              table_hbm.at[idx_v.at[j * RC + r]], bufs.at[b, r], gsem.at[b]
          )
          for r in range(RC)
      ]

    def out_copy(j, b):
      return pltpu.make_async_copy(
          bufs.at[b, :, :, pl.ds(0, D)],
          out_hbm.at[pl.ds(base_r + j * RC, RC)],
          osem.at[b],
      )

    for b in range(NB):
      for g in gathers(b, b):
        g.start()

    @pl.loop(0, CH - NB, step=NB)
    def _main(j0):
      for b in range(NB):
        j = j0 + b
        for g in gathers(j, b):
          g.wait()
        out_copy(j, b).start()
        out_copy(j, b).wait()
        for g in gathers(j + NB, b):
          g.start()

    for b in range(NB):
      j = CH - NB + b
      for g in gathers(j, b):
        g.wait()
      out_copy(j, b).start()
      out_copy(j, b).wait()

  return k


_k = _make()


@jax.jit
def kernel(input, table):
  tablep = jnp.pad(table, ((0, 0), (0, 64)))
  return _k(input, tablep)


# speedup vs baseline: 1.4574x; 1.4050x over previous
import functools

import jax
import jax.numpy as jnp
from jax import lax
from jax.experimental import pallas as pl
from jax.experimental.pallas import tpu as pltpu
from jax.experimental.pallas import tpu_sc as plsc

BATCH, HIST, D = 16384, 20, 64
NC, NS = 2, 16
NW = NC * NS
ROWS_W = BATCH // NW
RC = 8
CH = ROWS_W // RC
NB = 2


def _make():
  mesh = plsc.VectorSubcoreMesh(core_axis_name="c", subcore_axis_name="s")

  @functools.partial(
      pl.kernel,
      out_type=jax.ShapeDtypeStruct((BATCH, HIST, D), jnp.float32),
      mesh=mesh,
      scratch_types=[
          pltpu.VMEM((NB, RC, HIST), jnp.int32),
          pltpu.VMEM((NB, RC, HIST, D), jnp.float32),
          pltpu.SemaphoreType.DMA((NB,)),
          pltpu.SemaphoreType.DMA((NB,)),
          pltpu.SemaphoreType.DMA((NB,)),
      ],
      compiler_params=pltpu.CompilerParams(use_tc_tiling_on_sc=True),
  )
  def k(idx_hbm, table_hbm, out_hbm, idx_v, bufs, isem, gsem, osem):
    wid = lax.axis_index("s") * NC + lax.axis_index("c")
    base_r = wid * ROWS_W

    def idx_copy(j, b):
      return pltpu.make_async_copy(
          idx_hbm.at[pl.ds(base_r + j * RC, RC)], idx_v.at[b], isem.at[b]
      )

    def start_gathers(b):
      @pl.loop(0, RC)
      def _rows(rr):
        v0 = idx_v[b, rr, pl.ds(0, 16)]
        v1 = idx_v[b, rr, pl.ds(4, 16)]
        for h in range(HIST):
          idx = v0[h] if h < 16 else v1[h - 4]
          pltpu.async_copy(
              table_hbm.at[idx], bufs.at[b, rr, h], gsem.at[b]
          )

    def drain_gathers(j, b):
      pltpu.make_async_copy(
          out_hbm.at[pl.ds(base_r + j * RC, RC)], bufs.at[b], gsem.at[b]
      ).wait()

    def out_copy(j, b):
      return pltpu.make_async_copy(
          bufs.at[b], out_hbm.at[pl.ds(base_r + j * RC, RC)], osem.at[b]
      )

    for b in range(NB):
      idx_copy(b, b).start()
    for b in range(NB):
      idx_copy(b, b).wait()
      start_gathers(b)

    @pl.loop(0, CH - NB, step=NB)
    def _main(j0):
      for b in range(NB):
        j = j0 + b
        drain_gathers(j, b)
        out_copy(j, b).start()
        idx_copy(j + NB, b).start()
        out_copy(j, b).wait()
        idx_copy(j + NB, b).wait()
        start_gathers(b)

    for b in range(NB):
      j = CH - NB + b
      drain_gathers(j, b)
      out_copy(j, b).start()
      out_copy(j, b).wait()

  return k


_k = _make()


@jax.jit
def kernel(input, table):
  return _k(input, table)


# speedup vs baseline: 1.4584x; 1.0007x over previous
import functools

import jax
import jax.numpy as jnp
from jax import lax
from jax.experimental import pallas as pl
from jax.experimental.pallas import tpu as pltpu
from jax.experimental.pallas import tpu_sc as plsc

BATCH, HIST, D = 16384, 20, 64
NC, NS = 2, 16
NW = NC * NS
ROWS_W = BATCH // NW
RC = 8
CH = ROWS_W // RC
NB = 4


def _make():
  mesh = plsc.VectorSubcoreMesh(core_axis_name="c", subcore_axis_name="s")

  @functools.partial(
      pl.kernel,
      out_type=jax.ShapeDtypeStruct((BATCH, HIST, D), jnp.float32),
      mesh=mesh,
      scratch_types=[
          pltpu.VMEM((NB, RC, HIST), jnp.int32),
          pltpu.VMEM((NB, RC, HIST, D), jnp.float32),
          pltpu.SemaphoreType.DMA((NB,)),
          pltpu.SemaphoreType.DMA((NB,)),
          pltpu.SemaphoreType.DMA((NB,)),
      ],
      compiler_params=pltpu.CompilerParams(use_tc_tiling_on_sc=True),
  )
  def k(idx_hbm, table_hbm, out_hbm, idx_v, bufs, isem, gsem, osem):
    wid = lax.axis_index("s") * NC + lax.axis_index("c")
    base_r = wid * ROWS_W

    def idx_copy(j, b):
      return pltpu.make_async_copy(
          idx_hbm.at[pl.ds(base_r + j * RC, RC)], idx_v.at[b], isem.at[b]
      )

    def start_gathers(b):
      @pl.loop(0, RC)
      def _rows(rr):
        v0 = idx_v[b, rr, pl.ds(0, 16)]
        v1 = idx_v[b, rr, pl.ds(4, 16)]
        for h in range(HIST):
          idx = v0[h] if h < 16 else v1[h - 4]
          pltpu.async_copy(
              table_hbm.at[idx], bufs.at[b, rr, h], gsem.at[b]
          )

    def drain_gathers(j, b):
      pltpu.make_async_copy(
          out_hbm.at[pl.ds(base_r + j * RC, RC)], bufs.at[b], gsem.at[b]
      ).wait()

    def out_copy(j, b):
      return pltpu.make_async_copy(
          bufs.at[b], out_hbm.at[pl.ds(base_r + j * RC, RC)], osem.at[b]
      )

    for b in range(NB):
      idx_copy(b, b).start()
    for b in range(NB):
      idx_copy(b, b).wait()
      start_gathers(b)

    @pl.loop(0, CH - NB, step=NB)
    def _main(j0):
      for b in range(NB):
        j = j0 + b
        drain_gathers(j, b)
        out_copy(j, b).start()
        idx_copy(j + NB, b).start()
        out_copy(j, b).wait()
        idx_copy(j + NB, b).wait()
        start_gathers(b)

    for b in range(NB):
      j = CH - NB + b
      drain_gathers(j, b)
      out_copy(j, b).start()
      out_copy(j, b).wait()

  return k


_k = _make()


@jax.jit
def kernel(input, table):
  return _k(input, table)
